# Initial kernel scaffold; baseline (speedup 1.0000x reference)
import jax
import jax.numpy as jnp
from jax.experimental import pallas as pl


def kernel(node_features, edge_index, edge_features, W1_e, b1_e, W2_e, b2_e, W1_n, b1_n, W2_n, b2_n):
    raise NotImplementedError("write your pallas kernel here")


# speedup vs baseline: 1.6223x; 1.6223x over previous
import jax
import jax.numpy as jnp
from jax import lax
from jax.experimental import pallas as pl
from jax.experimental.pallas import tpu as pltpu
from jax.experimental.pallas import tpu_sc as plsc

N_NODES = 10000
N_EDGES = 320000
D = 128
ED = 16
H = 128

NC, NS = 2, 16
NW = NC * NS
E_PAD = 327680
EPW = E_PAD // NW
CH = 128
NCH = EPW // CH
K = 5
OUTER = NCH // K
SCH = 128
NSCH = EPW // SCH
ACC = 10240
DUMMY = N_NODES
NS_Z = 10
NZ = ACC // NS_Z
NWB = N_NODES // NS_Z

NODE_TILE = 2000
EDGE_TILE = 2048


def _node_proj_body(x_ref, ws_ref, wd_ref, p_ref, q_ref):
    x = x_ref[...]
    p_ref[...] = jnp.dot(x, ws_ref[...], preferred_element_type=jnp.float32)
    q_ref[...] = jnp.dot(x, wd_ref[...], preferred_element_type=jnp.float32)


def _node_proj(x, ws, wd):
    n_tiles = N_NODES // NODE_TILE
    return pl.pallas_call(
        _node_proj_body,
        grid=(n_tiles,),
        in_specs=[
            pl.BlockSpec((NODE_TILE, D), lambda t: (t, 0)),
            pl.BlockSpec((D, H), lambda t: (0, 0)),
            pl.BlockSpec((D, H), lambda t: (0, 0)),
        ],
        out_specs=[
            pl.BlockSpec((NODE_TILE, H), lambda t: (t, 0)),
            pl.BlockSpec((NODE_TILE, H), lambda t: (t, 0)),
        ],
        out_shape=[
            jax.ShapeDtypeStruct((N_NODES, H), jnp.float32),
            jax.ShapeDtypeStruct((N_NODES, H), jnp.float32),
        ],
    )(x, ws, wd)


def _gather_body(p_hbm, q_hbm, src_hbm, dstg_hbm, gs_hbm, gd_hbm,
                 idxs_v, idxd_v, rows_v, gsem, wsem):
    c = lax.axis_index("c")
    s = lax.axis_index("s")
    wid = s * NC + c
    ebase = wid * EPW
    pltpu.sync_copy(src_hbm.at[pl.ds(ebase, EPW)], idxs_v)
    pltpu.sync_copy(dstg_hbm.at[pl.ds(ebase, EPW)], idxd_v)

    def run(tab, idx_v, out_hbm):
        def outer(o, carry):
            j0 = o * K
            hs = [pltpu.async_copy(
                      tab.at[idx_v.at[pl.ds((j0 + b) * CH, CH)]],
                      rows_v.at[b], gsem)
                  for b in range(K)]
            for h_ in hs:
                h_.wait()
            ws_ = [pltpu.async_copy(
                       rows_v.at[b],
                       out_hbm.at[pl.ds(ebase + (j0 + b) * CH, CH)], wsem)
                   for b in range(K)]
            for w_ in ws_:
                w_.wait()
            return carry
        lax.fori_loop(0, OUTER, outer, 0)

    run(p_hbm, idxs_v, gs_hbm)
    run(q_hbm, idxd_v, gd_hbm)


_gather = pl.kernel(
    _gather_body,
    out_type=(
        jax.ShapeDtypeStruct((E_PAD, D), jnp.float32),
        jax.ShapeDtypeStruct((E_PAD, D), jnp.float32),
    ),
    mesh=plsc.VectorSubcoreMesh(core_axis_name="c", subcore_axis_name="s"),
    compiler_params=pltpu.CompilerParams(use_tc_tiling_on_sc=False),
    scratch_types=[
        pltpu.VMEM((EPW,), jnp.int32),
        pltpu.VMEM((EPW,), jnp.int32),
        pltpu.VMEM((K, CH, D), jnp.float32),
        pltpu.SemaphoreType.DMA,
        pltpu.SemaphoreType.DMA,
    ],
)


def _edge_mlp_body(gs_ref, gd_ref, ef_ref, w1m_ref, b1_ref, w2_ref, b2_ref,
                   out_ref):
    pre = (gs_ref[...] + gd_ref[...]
           + jnp.dot(ef_ref[...], w1m_ref[...],
                     preferred_element_type=jnp.float32)
           + b1_ref[...])
    h = jnp.maximum(pre, 0.0)
    out_ref[...] = (jnp.dot(h, w2_ref[...], preferred_element_type=jnp.float32)
                    + b2_ref[...])


def _edge_mlp(gs, gd, ef, w1m, b1, w2, b2):
    n_tiles = E_PAD // EDGE_TILE
    return pl.pallas_call(
        _edge_mlp_body,
        grid=(n_tiles,),
        in_specs=[
            pl.BlockSpec((EDGE_TILE, H), lambda t: (t, 0)),
            pl.BlockSpec((EDGE_TILE, H), lambda t: (t, 0)),
            pl.BlockSpec((EDGE_TILE, ED), lambda t: (t, 0)),
            pl.BlockSpec((ED, H), lambda t: (0, 0)),
            pl.BlockSpec((1, H), lambda t: (0, 0)),
            pl.BlockSpec((H, ED), lambda t: (0, 0)),
            pl.BlockSpec((1, ED), lambda t: (0, 0)),
        ],
        out_specs=pl.BlockSpec((EDGE_TILE, ED), lambda t: (t, 0)),
        out_shape=jax.ShapeDtypeStruct((E_PAD, ED), jnp.float32),
    )(gs, gd, ef, w1m, b1, w2, b2)


def _segsum_body(eupd_hbm, dsts_hbm, out_hbm, acc_shr, idx_v, rows_v, zbuf):
    c = lax.axis_index("c")
    s = lax.axis_index("s")
    wid = s * NC + c

    def zrow(i, carry):
        zbuf[i] = jnp.zeros((ED,), jnp.float32)
        return carry
    lax.fori_loop(0, NZ, zrow, 0)

    @pl.when(s < NS_Z)
    def _():
        pltpu.sync_copy(zbuf, acc_shr.at[pl.ds(s * NZ, NZ)])
    plsc.subcore_barrier()

    pltpu.sync_copy(dsts_hbm.at[pl.ds(wid * EPW, EPW)], idx_v)

    def chunk(j, carry):
        pltpu.sync_copy(eupd_hbm.at[pl.ds(wid * EPW + j * SCH, SCH)], rows_v)
        pltpu.sync_copy(rows_v, acc_shr.at[idx_v.at[pl.ds(j * SCH, SCH)]],
                        add=True)
        return carry
    lax.fori_loop(0, NSCH, chunk, 0)
    plsc.subcore_barrier()

    @pl.when(s < NS_Z)
    def _():
        pltpu.sync_copy(acc_shr.at[pl.ds(s * NWB, NWB)],
                        zbuf.at[pl.ds(0, NWB)])
        pltpu.sync_copy(zbuf.at[pl.ds(0, NWB)],
                        out_hbm.at[pl.ds(c * N_NODES + s * NWB, NWB)])


_segsum = pl.kernel(
    _segsum_body,
    out_type=jax.ShapeDtypeStruct((NC * N_NODES, ED), jnp.float32),
    mesh=plsc.VectorSubcoreMesh(core_axis_name="c", subcore_axis_name="s"),
    compiler_params=pltpu.CompilerParams(use_tc_tiling_on_sc=False),
    scratch_types=[
        pltpu.VMEM_SHARED((ACC, ED), jnp.float32),
        pltpu.VMEM((EPW,), jnp.int32),
        pltpu.VMEM((SCH, ED), jnp.float32),
        pltpu.VMEM((NZ, ED), jnp.float32),
    ],
)


def _node_mlp_body(x_ref, e0_ref, e1_ref, w1x_ref, w1e_ref, b1_ref, w2_ref,
                   b2_ref, out_ref):
    es = e0_ref[...] + e1_ref[...]
    h = jnp.maximum(
        jnp.dot(x_ref[...], w1x_ref[...], preferred_element_type=jnp.float32)
        + jnp.dot(es, w1e_ref[...], preferred_element_type=jnp.float32)
        + b1_ref[...], 0.0)
    out_ref[...] = (jnp.dot(h, w2_ref[...], preferred_element_type=jnp.float32)
                    + b2_ref[...])


def _node_mlp(x, eparts, w1x, w1e, b1, w2, b2):
    n_tiles = N_NODES // NODE_TILE
    half = N_NODES // NODE_TILE
    return pl.pallas_call(
        _node_mlp_body,
        grid=(n_tiles,),
        in_specs=[
            pl.BlockSpec((NODE_TILE, D), lambda t: (t, 0)),
            pl.BlockSpec((NODE_TILE, ED), lambda t: (t, 0)),
            pl.BlockSpec((NODE_TILE, ED), lambda t: (t + half, 0)),
            pl.BlockSpec((D, H), lambda t: (0, 0)),
            pl.BlockSpec((ED, H), lambda t: (0, 0)),
            pl.BlockSpec((1, H), lambda t: (0, 0)),
            pl.BlockSpec((H, D), lambda t: (0, 0)),
            pl.BlockSpec((1, D), lambda t: (0, 0)),
        ],
        out_specs=pl.BlockSpec((NODE_TILE, D), lambda t: (t, 0)),
        out_shape=jax.ShapeDtypeStruct((N_NODES, D), jnp.float32),
    )(x, eparts, eparts, w1x, w1e, b1, w2, b2)


def kernel(node_features, edge_index, edge_features,
           W1_e, b1_e, W2_e, b2_e, W1_n, b1_n, W2_n, b2_n):
    src = edge_index[0].astype(jnp.int32)
    dst = edge_index[1].astype(jnp.int32)
    npad = E_PAD - N_EDGES
    src_p = jnp.concatenate([src, jnp.zeros((npad,), jnp.int32)])
    dstg_p = jnp.concatenate([dst, jnp.zeros((npad,), jnp.int32)])
    dsts_p = jnp.concatenate([dst, jnp.full((npad,), DUMMY, jnp.int32)])
    ef_p = jnp.concatenate(
        [edge_features, jnp.zeros((npad, ED), jnp.float32)])

    w1s = W1_e[:D]
    w1m = W1_e[D:D + ED]
    w1d = W1_e[D + ED:]
    w1x = W1_n[:D]
    w1e = W1_n[D:]
    b1e = b1_e.reshape(1, H)
    b2e = b2_e.reshape(1, ED)
    b1n = b1_n.reshape(1, H)
    b2n = b2_n.reshape(1, D)

    p, q = _node_proj(node_features, w1s, w1d)
    gs, gd = _gather(p, q, src_p, dstg_p)
    e_up_pad = _edge_mlp(gs, gd, ef_p, w1m, b1e, W2_e, b2e)
    eparts = _segsum(e_up_pad, dsts_p)
    h_updated = _node_mlp(node_features, eparts, w1x, w1e, b1n, W2_n, b2n)
    e_updated = e_up_pad[:N_EDGES]
    return (h_updated, e_updated)
